# baseline (speedup 1.0000x reference)
import functools

import jax
import jax.numpy as jnp
from jax import lax
from jax.experimental import pallas as pl
from jax.experimental.pallas import tpu as pltpu
from jax.experimental.pallas import tpu_sc as plsc

B, S, D = 1, 2048, 1024
H, KV, HD = 16, 8, 64
E, TOPK, FF = 8, 2, 2048
EPS = 1e-6
THETA = 1e6
SB = 256
SCALE = 1.0 / (HD ** 0.5)
BLK = 128
NSLOT = 5120
G = NSLOT // BLK
GPAD = 128
CHA = 32
CHB = 32

_NC, _NS = 2, 16
NW = _NC * _NS


def _qkv_body(x_ref, ln1_ref, wq_ref, wk_ref, wv_ref, q_ref, k_ref, v_ref):
    i = pl.program_id(0)
    x = x_ref[...]
    var = jnp.mean(x * x, axis=1, keepdims=True)
    xb = (x * lax.rsqrt(var + EPS) * ln1_ref[...]).astype(jnp.bfloat16)
    q = jnp.dot(xb, wq_ref[...], preferred_element_type=jnp.float32)
    k = jnp.dot(xb, wk_ref[...], preferred_element_type=jnp.float32)
    v = jnp.dot(xb, wv_ref[...], preferred_element_type=jnp.float32)
    half = HD // 2
    t = (i * SB + lax.broadcasted_iota(jnp.int32, (SB, half), 0)).astype(jnp.float32)
    j2 = lax.broadcasted_iota(jnp.int32, (SB, half), 1).astype(jnp.float32)
    inv = jnp.exp(j2 * (-jnp.log(jnp.float32(THETA)) / half))
    fr = t * inv
    cosf = jnp.concatenate([jnp.cos(fr), jnp.cos(fr)], axis=1)
    sinf = jnp.concatenate([jnp.sin(fr), jnp.sin(fr)], axis=1)
    for h in range(H):
        qh = q[:, h * HD:(h + 1) * HD]
        qr = jnp.concatenate([-qh[:, half:], qh[:, :half]], axis=1)
        q_ref[h] = (qh * cosf + qr * sinf).astype(jnp.bfloat16)
    for h in range(KV):
        kh = k[:, h * HD:(h + 1) * HD]
        kr = jnp.concatenate([-kh[:, half:], kh[:, :half]], axis=1)
        k_ref[h] = (kh * cosf + kr * sinf).astype(jnp.bfloat16)
        v_ref[h] = v[:, h * HD:(h + 1) * HD].astype(jnp.bfloat16)


def _qkv(x, ln1, wq, wk, wv):
    return pl.pallas_call(
        _qkv_body,
        grid=(S // SB,),
        in_specs=[
            pl.BlockSpec((SB, D), lambda i: (i, 0)),
            pl.BlockSpec((1, D), lambda i: (0, 0)),
            pl.BlockSpec((D, H * HD), lambda i: (0, 0)),
            pl.BlockSpec((D, KV * HD), lambda i: (0, 0)),
            pl.BlockSpec((D, KV * HD), lambda i: (0, 0)),
        ],
        out_specs=[
            pl.BlockSpec((H, SB, HD), lambda i: (0, i, 0)),
            pl.BlockSpec((KV, SB, HD), lambda i: (0, i, 0)),
            pl.BlockSpec((KV, SB, HD), lambda i: (0, i, 0)),
        ],
        out_shape=[
            jax.ShapeDtypeStruct((H, S, HD), jnp.bfloat16),
            jax.ShapeDtypeStruct((KV, S, HD), jnp.bfloat16),
            jax.ShapeDtypeStruct((KV, S, HD), jnp.bfloat16),
        ],
    )(x, ln1, wq, wk, wv)


def _attn_step(q, kj, vj, m, l, acc, masked):
    s = lax.dot_general(q, kj, (((1,), (1,)), ((), ())),
                        preferred_element_type=jnp.float32) * SCALE
    if masked:
        r = lax.rem(lax.broadcasted_iota(jnp.int32, (2 * SB, SB), 0), SB)
        c = lax.broadcasted_iota(jnp.int32, (2 * SB, SB), 1)
        s = jnp.where(r >= c, s, -jnp.inf)
    mnew = jnp.maximum(m, jnp.max(s, axis=1, keepdims=True))
    p = jnp.exp(s - mnew)
    corr = jnp.exp(m - mnew)
    lnew = l * corr + jnp.sum(p, axis=1, keepdims=True)
    accnew = acc * corr + jnp.dot(p.astype(jnp.bfloat16), vj,
                                  preferred_element_type=jnp.float32)
    return mnew, lnew, accnew


def _attn_body(q_ref, k_ref, v_ref, o_ref):
    i = pl.program_id(1)
    q = jnp.concatenate([q_ref[0], q_ref[1]], axis=0)

    def step(j, carry):
        m, l, acc = carry
        kj = k_ref[0, pl.ds(j * SB, SB), :]
        vj = v_ref[0, pl.ds(j * SB, SB), :]
        return _attn_step(q, kj, vj, m, l, acc, masked=False)

    m0 = jnp.full((2 * SB, 1), -jnp.inf, jnp.float32)
    l0 = jnp.zeros((2 * SB, 1), jnp.float32)
    a0 = jnp.zeros((2 * SB, HD), jnp.float32)
    m, l, acc = lax.fori_loop(0, i, step, (m0, l0, a0))
    kj = k_ref[0, pl.ds(i * SB, SB), :]
    vj = v_ref[0, pl.ds(i * SB, SB), :]
    m, l, acc = _attn_step(q, kj, vj, m, l, acc, masked=True)
    o = (acc / l).astype(jnp.bfloat16)
    o_ref[0] = o[:SB]
    o_ref[1] = o[SB:]


def _attention(q, k, v):
    return pl.pallas_call(
        _attn_body,
        grid=(KV, S // SB),
        in_specs=[
            pl.BlockSpec((H // KV, SB, HD), lambda h, i: (h, i, 0)),
            pl.BlockSpec((1, S, HD), lambda h, i: (h, 0, 0)),
            pl.BlockSpec((1, S, HD), lambda h, i: (h, 0, 0)),
        ],
        out_specs=pl.BlockSpec((H // KV, SB, HD), lambda h, i: (h, i, 0)),
        out_shape=jax.ShapeDtypeStruct((H, S, HD), jnp.bfloat16),
    )(q, k, v)


def _oproj_body(a_ref, wo_ref, res_ref, o_ref):
    acc = res_ref[...]
    for h in range(H):
        acc = acc + jnp.dot(a_ref[h], wo_ref[h],
                            preferred_element_type=jnp.float32)
    o_ref[...] = acc


def _oproj(attn, wo, res):
    return pl.pallas_call(
        _oproj_body,
        grid=(S // SB,),
        in_specs=[
            pl.BlockSpec((H, SB, HD), lambda i: (0, i, 0)),
            pl.BlockSpec((H, HD, D), lambda i: (0, 0, 0)),
            pl.BlockSpec((SB, D), lambda i: (i, 0)),
        ],
        out_specs=pl.BlockSpec((SB, D), lambda i: (i, 0)),
        out_shape=jax.ShapeDtypeStruct((S, D), jnp.float32),
    )(attn, wo.reshape(H, HD, D), res)


_CR = 512


def _cast_body(x_ref, o_ref):
    o_ref[...] = x_ref[...].astype(jnp.bfloat16)


def _cast_bf16(x, shape):
    flat = x.reshape(-1, FF)
    n = flat.shape[0]
    out = pl.pallas_call(
        _cast_body,
        grid=(n // _CR,),
        in_specs=[pl.BlockSpec((_CR, FF), lambda i: (i, 0))],
        out_specs=pl.BlockSpec((_CR, FF), lambda i: (i, 0)),
        out_shape=jax.ShapeDtypeStruct((n, FF), jnp.bfloat16),
    )(flat)
    return out.reshape(shape)


def _route_body(h_ref, ln2_ref, gw_ref,
                x2_ref, lg_ref, p0_ref, p1_ref, w0_ref, w1_ref, be_ref):
    hx = h_ref[...]
    var = jnp.mean(hx * hx, axis=1, keepdims=True)
    x2 = hx * lax.rsqrt(var + EPS) * ln2_ref[...]
    x2_ref[...] = x2
    logits = jnp.dot(x2, gw_ref[...], preferred_element_type=jnp.float32)
    lg_ref[...] = logits
    mx = jnp.max(logits, axis=1, keepdims=True)
    ex = jnp.exp(logits - mx)
    probs = ex / jnp.sum(ex, axis=1, keepdims=True)
    lane = lax.broadcasted_iota(jnp.int32, (S, E), 1)
    v1 = jnp.max(probs, axis=1, keepdims=True)
    i1 = jnp.min(jnp.where(probs == v1, lane, E), axis=1, keepdims=True)
    probs2 = jnp.where(lane == i1, -1.0, probs)
    v2 = jnp.max(probs2, axis=1, keepdims=True)
    i2 = jnp.min(jnp.where(probs2 == v2, lane, E), axis=1, keepdims=True)
    tot = v1 + v2
    w0_ref[...] = v1 / tot
    w1_ref[...] = v2 / tot
    oh0 = (lane == i1).astype(jnp.float32)
    oh1 = (lane == i2).astype(jnp.float32)
    CH = 256
    tril = (lax.broadcasted_iota(jnp.int32, (CH, CH), 1)
            < lax.broadcasted_iota(jnp.int32, (CH, CH), 0)).astype(jnp.float32)
    base = jnp.zeros((1, E), jnp.float32)
    ranks = []
    for oh in (oh0, oh1):
        parts = []
        for c in range(S // CH):
            blk = oh[c * CH:(c + 1) * CH]
            parts.append(jnp.dot(tril, blk, preferred_element_type=jnp.float32)
                         + base)
            base = base + jnp.sum(blk, axis=0, keepdims=True)
        ranks.append(jnp.concatenate(parts, axis=0))
    cnt = base
    padded = jnp.ceil(cnt / BLK) * BLK
    sut = (lax.broadcasted_iota(jnp.int32, (E, E), 0)
           < lax.broadcasted_iota(jnp.int32, (E, E), 1)).astype(jnp.float32)
    base_e = jnp.dot(padded, sut, preferred_element_type=jnp.float32)
    pos0 = jnp.sum((base_e + ranks[0]) * oh0, axis=1, keepdims=True)
    pos1 = jnp.sum((base_e + ranks[1]) * oh1, axis=1, keepdims=True)
    p0_ref[...] = pos0.astype(jnp.int32)
    p1_ref[...] = pos1.astype(jnp.int32)
    ends = base_e + padded
    bs = lax.broadcasted_iota(jnp.int32, (GPAD, 1), 0).astype(jnp.float32) * BLK
    be = jnp.sum((bs >= ends).astype(jnp.int32), axis=1, keepdims=True)
    be_ref[...] = jnp.minimum(be, E - 1)


def _route(hidden, ln2, gate_w):
    return pl.pallas_call(
        _route_body,
        grid=(1,),
        in_specs=[
            pl.BlockSpec((S, D), lambda i: (0, 0)),
            pl.BlockSpec((1, D), lambda i: (0, 0)),
            pl.BlockSpec((D, E), lambda i: (0, 0)),
        ],
        out_specs=[
            pl.BlockSpec((S, D), lambda i: (0, 0)),
            pl.BlockSpec((S, E), lambda i: (0, 0)),
            pl.BlockSpec((S, 1), lambda i: (0, 0)),
            pl.BlockSpec((S, 1), lambda i: (0, 0)),
            pl.BlockSpec((S, 1), lambda i: (0, 0)),
            pl.BlockSpec((S, 1), lambda i: (0, 0)),
            pl.BlockSpec((GPAD, 1), lambda i: (0, 0)),
        ],
        out_shape=[
            jax.ShapeDtypeStruct((S, D), jnp.float32),
            jax.ShapeDtypeStruct((S, E), jnp.float32),
            jax.ShapeDtypeStruct((S, 1), jnp.int32),
            jax.ShapeDtypeStruct((S, 1), jnp.int32),
            jax.ShapeDtypeStruct((S, 1), jnp.float32),
            jax.ShapeDtypeStruct((S, 1), jnp.float32),
            jax.ShapeDtypeStruct((GPAD, 1), jnp.int32),
        ],
    )(hidden, ln2, gate_w)


def _dispatch(x2, posf):
    mesh = plsc.VectorSubcoreMesh(core_axis_name="c", subcore_axis_name="s")
    apt = (TOPK * S) // NW

    @functools.partial(
        pl.kernel, mesh=mesh,
        out_type=jax.ShapeDtypeStruct((NSLOT, D), jnp.float32),
        scratch_types=[
            pltpu.VMEM((CHA,), jnp.int32),
            pltpu.VMEM((CHA, D), jnp.float32),
            pltpu.SemaphoreType.DMA,
        ],
    )
    def k(x2_hbm, pos_hbm, xs_hbm, idx_v, rows_v, sem):
        wid = lax.axis_index("s") * _NC + lax.axis_index("c")
        for c in range(apt // CHA):
            a0 = wid * apt + c * CHA
            t0 = lax.rem(a0, S)
            pltpu.sync_copy(x2_hbm.at[pl.ds(t0, CHA)], rows_v)
            pltpu.sync_copy(pos_hbm.at[pl.ds(a0, CHA)], idx_v)
            pltpu.async_copy(rows_v, xs_hbm.at[idx_v], sem).wait()

    return k(x2, posf)


def _mlp_body(be_ref, x_ref, w1_ref, w3_ref, w2_ref, y_ref):
    x = x_ref[...].astype(jnp.bfloat16)
    a = jnp.dot(x, w1_ref[0], preferred_element_type=jnp.float32)
    g = jnp.dot(x, w3_ref[0], preferred_element_type=jnp.float32)
    hcur = (a * jax.nn.sigmoid(a) * g).astype(jnp.bfloat16)
    y_ref[...] = jnp.dot(hcur, w2_ref[0], preferred_element_type=jnp.float32)


def _mlp(be, xs, w1, w3, w2):
    grid_spec = pltpu.PrefetchScalarGridSpec(
        num_scalar_prefetch=1,
        grid=(G,),
        in_specs=[
            pl.BlockSpec((BLK, D), lambda b, be: (b, 0)),
            pl.BlockSpec((1, D, FF), lambda b, be: (be[b], 0, 0)),
            pl.BlockSpec((1, D, FF), lambda b, be: (be[b], 0, 0)),
            pl.BlockSpec((1, FF, D), lambda b, be: (be[b], 0, 0)),
        ],
        out_specs=pl.BlockSpec((BLK, D), lambda b, be: (b, 0)),
    )
    return pl.pallas_call(
        _mlp_body,
        grid_spec=grid_spec,
        out_shape=jax.ShapeDtypeStruct((NSLOT, D), jnp.float32),
    )(be, xs, w1, w3, w2)


def _combine_gather(ys, p0, p1):
    mesh = plsc.VectorSubcoreMesh(core_axis_name="c", subcore_axis_name="s")
    tpt = S // NW

    @functools.partial(
        pl.kernel, mesh=mesh,
        out_type=(jax.ShapeDtypeStruct((S, D), jnp.float32),
                  jax.ShapeDtypeStruct((S, D), jnp.float32)),
        scratch_types=[
            pltpu.VMEM((CHB,), jnp.int32),
            pltpu.VMEM((CHB, D), jnp.float32),
            pltpu.SemaphoreType.DMA,
        ],
    )
    def k(ys_hbm, p0_hbm, p1_hbm, g0_hbm, g1_hbm, idx_v, rows_v, sem):
        wid = lax.axis_index("s") * _NC + lax.axis_index("c")
        for c in range(tpt // CHB):
            t0 = wid * tpt + c * CHB
            pltpu.sync_copy(p0_hbm.at[pl.ds(t0, CHB)], idx_v)
            pltpu.async_copy(ys_hbm.at[idx_v], rows_v, sem).wait()
            pltpu.sync_copy(rows_v, g0_hbm.at[pl.ds(t0, CHB)])
            pltpu.sync_copy(p1_hbm.at[pl.ds(t0, CHB)], idx_v)
            pltpu.async_copy(ys_hbm.at[idx_v], rows_v, sem).wait()
            pltpu.sync_copy(rows_v, g1_hbm.at[pl.ds(t0, CHB)])

    return k(ys, p0, p1)


def _final_body(h_ref, w0_ref, g0_ref, w1_ref, g1_ref, o_ref):
    o_ref[...] = (h_ref[...] + w0_ref[...] * g0_ref[...]
                  + w1_ref[...] * g1_ref[...])


def _final(hidden, w0, g0, w1, g1):
    return pl.pallas_call(
        _final_body,
        grid=(S // SB,),
        in_specs=[
            pl.BlockSpec((SB, D), lambda i: (i, 0)),
            pl.BlockSpec((SB, 1), lambda i: (i, 0)),
            pl.BlockSpec((SB, D), lambda i: (i, 0)),
            pl.BlockSpec((SB, 1), lambda i: (i, 0)),
            pl.BlockSpec((SB, D), lambda i: (i, 0)),
        ],
        out_specs=pl.BlockSpec((SB, D), lambda i: (i, 0)),
        out_shape=jax.ShapeDtypeStruct((S, D), jnp.float32),
    )(hidden, w0, g0, w1, g1)


def kernel(hidden_state, ln1_w, ln2_w, wq, wk, wv, wo, gate_w, w1, w3, w2):
    x = hidden_state.reshape(S, D)
    q, k, v = _qkv(x, ln1_w.reshape(1, D),
                   wq.astype(jnp.bfloat16), wk.astype(jnp.bfloat16),
                   wv.astype(jnp.bfloat16))
    attn = _attention(q, k, v)
    hidden = _oproj(attn, wo.astype(jnp.bfloat16), x)
    x2, logits, p0, p1, w0c, w1c, be = _route(hidden, ln2_w.reshape(1, D),
                                              gate_w)
    posf = jnp.concatenate([p0[:, 0], p1[:, 0]], axis=0)
    xs = _dispatch(x2, posf)
    ys = _mlp(be[:G, 0], xs, _cast_bf16(w1, w1.shape),
              _cast_bf16(w3, w3.shape), _cast_bf16(w2, w2.shape))
    g0, g1 = _combine_gather(ys, p0[:, 0], p1[:, 0])
    out = _final(hidden, w0c, g0, w1c, g1)
    return out.reshape(B, S, D), logits.reshape(B, S, E)


# speedup vs baseline: 1.2403x; 1.2403x over previous
import functools

import jax
import jax.numpy as jnp
from jax import lax
from jax.experimental import pallas as pl
from jax.experimental.pallas import tpu as pltpu
from jax.experimental.pallas import tpu_sc as plsc

B, S, D = 1, 2048, 1024
H, KV, HD = 16, 8, 64
E, TOPK, FF = 8, 2, 2048
EPS = 1e-6
THETA = 1e6
SB = 256
SCALE = 1.0 / (HD ** 0.5)
BLK = 128
NSLOT = 5120
G = NSLOT // BLK
GPAD = 128
CHA = 32
CHB = 32

_NC, _NS = 2, 16
NW = _NC * _NS


def _qkv_body(x_ref, ln1_ref, wq_ref, wk_ref, wv_ref, q_ref, k_ref, v_ref):
    i = pl.program_id(0)
    x = x_ref[...]
    var = jnp.mean(x * x, axis=1, keepdims=True)
    xb = (x * lax.rsqrt(var + EPS) * ln1_ref[...]).astype(jnp.bfloat16)
    q = jnp.dot(xb, wq_ref[...], preferred_element_type=jnp.float32)
    k = jnp.dot(xb, wk_ref[...], preferred_element_type=jnp.float32)
    v = jnp.dot(xb, wv_ref[...], preferred_element_type=jnp.float32)
    half = HD // 2
    t = (i * SB + lax.broadcasted_iota(jnp.int32, (SB, half), 0)).astype(jnp.float32)
    j2 = lax.broadcasted_iota(jnp.int32, (SB, half), 1).astype(jnp.float32)
    inv = jnp.exp(j2 * (-jnp.log(jnp.float32(THETA)) / half))
    fr = t * inv
    cosf = jnp.concatenate([jnp.cos(fr), jnp.cos(fr)], axis=1)
    sinf = jnp.concatenate([jnp.sin(fr), jnp.sin(fr)], axis=1)
    for h in range(H):
        qh = q[:, h * HD:(h + 1) * HD]
        qr = jnp.concatenate([-qh[:, half:], qh[:, :half]], axis=1)
        q_ref[h] = (qh * cosf + qr * sinf).astype(jnp.bfloat16)
    for h in range(KV):
        kh = k[:, h * HD:(h + 1) * HD]
        kr = jnp.concatenate([-kh[:, half:], kh[:, :half]], axis=1)
        k_ref[h] = (kh * cosf + kr * sinf).astype(jnp.bfloat16)
        v_ref[h] = v[:, h * HD:(h + 1) * HD].astype(jnp.bfloat16)


def _qkv(x, ln1, wq, wk, wv):
    return pl.pallas_call(
        _qkv_body,
        grid=(S // SB,),
        in_specs=[
            pl.BlockSpec((SB, D), lambda i: (i, 0)),
            pl.BlockSpec((1, D), lambda i: (0, 0)),
            pl.BlockSpec((D, H * HD), lambda i: (0, 0)),
            pl.BlockSpec((D, KV * HD), lambda i: (0, 0)),
            pl.BlockSpec((D, KV * HD), lambda i: (0, 0)),
        ],
        out_specs=[
            pl.BlockSpec((H, SB, HD), lambda i: (0, i, 0)),
            pl.BlockSpec((KV, SB, HD), lambda i: (0, i, 0)),
            pl.BlockSpec((KV, SB, HD), lambda i: (0, i, 0)),
        ],
        out_shape=[
            jax.ShapeDtypeStruct((H, S, HD), jnp.bfloat16),
            jax.ShapeDtypeStruct((KV, S, HD), jnp.bfloat16),
            jax.ShapeDtypeStruct((KV, S, HD), jnp.bfloat16),
        ],
    )(x, ln1, wq, wk, wv)


def _attn_step(q, kj, vj, m, l, acc, masked):
    s = lax.dot_general(q, kj, (((1,), (1,)), ((), ())),
                        preferred_element_type=jnp.float32) * SCALE
    if masked:
        r = lax.rem(lax.broadcasted_iota(jnp.int32, (2 * SB, SB), 0), SB)
        c = lax.broadcasted_iota(jnp.int32, (2 * SB, SB), 1)
        s = jnp.where(r >= c, s, -jnp.inf)
    mnew = jnp.maximum(m, jnp.max(s, axis=1, keepdims=True))
    p = jnp.exp(s - mnew)
    corr = jnp.exp(m - mnew)
    lnew = l * corr + jnp.sum(p, axis=1, keepdims=True)
    accnew = acc * corr + jnp.dot(p.astype(jnp.bfloat16), vj,
                                  preferred_element_type=jnp.float32)
    return mnew, lnew, accnew


def _attn_body(q_ref, k_ref, v_ref, o_ref):
    i = pl.program_id(1)
    q = jnp.concatenate([q_ref[0], q_ref[1]], axis=0)

    def step(j, carry):
        m, l, acc = carry
        kj = k_ref[0, pl.ds(j * SB, SB), :]
        vj = v_ref[0, pl.ds(j * SB, SB), :]
        return _attn_step(q, kj, vj, m, l, acc, masked=False)

    m0 = jnp.full((2 * SB, 1), -jnp.inf, jnp.float32)
    l0 = jnp.zeros((2 * SB, 1), jnp.float32)
    a0 = jnp.zeros((2 * SB, HD), jnp.float32)
    m, l, acc = lax.fori_loop(0, i, step, (m0, l0, a0))
    kj = k_ref[0, pl.ds(i * SB, SB), :]
    vj = v_ref[0, pl.ds(i * SB, SB), :]
    m, l, acc = _attn_step(q, kj, vj, m, l, acc, masked=True)
    o = (acc / l).astype(jnp.bfloat16)
    o_ref[...] = jnp.concatenate([o[:SB], o[SB:]], axis=1)


def _attention(q, k, v):
    return pl.pallas_call(
        _attn_body,
        grid=(KV, S // SB),
        in_specs=[
            pl.BlockSpec((H // KV, SB, HD), lambda h, i: (h, i, 0)),
            pl.BlockSpec((1, S, HD), lambda h, i: (h, 0, 0)),
            pl.BlockSpec((1, S, HD), lambda h, i: (h, 0, 0)),
        ],
        out_specs=pl.BlockSpec((SB, (H // KV) * HD), lambda h, i: (i, h)),
        out_shape=jax.ShapeDtypeStruct((S, H * HD), jnp.bfloat16),
    )(q, k, v)


def _oproj_body(a_ref, wo_ref, res_ref, o_ref):
    o_ref[...] = res_ref[...] + jnp.dot(a_ref[...], wo_ref[...],
                                        preferred_element_type=jnp.float32)


def _oproj(attn, wo, res):
    return pl.pallas_call(
        _oproj_body,
        grid=(S // SB,),
        in_specs=[
            pl.BlockSpec((SB, H * HD), lambda i: (i, 0)),
            pl.BlockSpec((H * HD, D), lambda i: (0, 0)),
            pl.BlockSpec((SB, D), lambda i: (i, 0)),
        ],
        out_specs=pl.BlockSpec((SB, D), lambda i: (i, 0)),
        out_shape=jax.ShapeDtypeStruct((S, D), jnp.float32),
    )(attn, wo, res)


_CR = 1024


def _cast_body(x_ref, o_ref):
    o_ref[0] = x_ref[0].astype(jnp.bfloat16)


def _cast_bf16(x):
    e, r, c = x.shape
    return pl.pallas_call(
        _cast_body,
        grid=(e, r // _CR),
        in_specs=[pl.BlockSpec((1, _CR, c), lambda i, j: (i, j, 0))],
        out_specs=pl.BlockSpec((1, _CR, c), lambda i, j: (i, j, 0)),
        out_shape=jax.ShapeDtypeStruct((e, r, c), jnp.bfloat16),
    )(x)


def _route_body(h_ref, ln2_ref, gw_ref,
                x2_ref, lg_ref, p0_ref, p1_ref, w0_ref, w1_ref, be_ref):
    hx = h_ref[...]
    var = jnp.mean(hx * hx, axis=1, keepdims=True)
    x2 = hx * lax.rsqrt(var + EPS) * ln2_ref[...]
    x2_ref[...] = x2
    logits = jnp.dot(x2, gw_ref[...], preferred_element_type=jnp.float32)
    lg_ref[...] = logits
    mx = jnp.max(logits, axis=1, keepdims=True)
    ex = jnp.exp(logits - mx)
    probs = ex / jnp.sum(ex, axis=1, keepdims=True)
    lane = lax.broadcasted_iota(jnp.int32, (S, E), 1)
    v1 = jnp.max(probs, axis=1, keepdims=True)
    i1 = jnp.min(jnp.where(probs == v1, lane, E), axis=1, keepdims=True)
    probs2 = jnp.where(lane == i1, -1.0, probs)
    v2 = jnp.max(probs2, axis=1, keepdims=True)
    i2 = jnp.min(jnp.where(probs2 == v2, lane, E), axis=1, keepdims=True)
    tot = v1 + v2
    w0_ref[...] = v1 / tot
    w1_ref[...] = v2 / tot
    oh0 = (lane == i1).astype(jnp.float32)
    oh1 = (lane == i2).astype(jnp.float32)
    CH = 256
    tril = (lax.broadcasted_iota(jnp.int32, (CH, CH), 1)
            < lax.broadcasted_iota(jnp.int32, (CH, CH), 0)).astype(jnp.float32)
    base = jnp.zeros((1, E), jnp.float32)
    ranks = []
    for oh in (oh0, oh1):
        parts = []
        for c in range(S // CH):
            blk = oh[c * CH:(c + 1) * CH]
            parts.append(jnp.dot(tril, blk, preferred_element_type=jnp.float32)
                         + base)
            base = base + jnp.sum(blk, axis=0, keepdims=True)
        ranks.append(jnp.concatenate(parts, axis=0))
    cnt = base
    padded = jnp.ceil(cnt / BLK) * BLK
    sut = (lax.broadcasted_iota(jnp.int32, (E, E), 0)
           < lax.broadcasted_iota(jnp.int32, (E, E), 1)).astype(jnp.float32)
    base_e = jnp.dot(padded, sut, preferred_element_type=jnp.float32)
    pos0 = jnp.sum((base_e + ranks[0]) * oh0, axis=1, keepdims=True)
    pos1 = jnp.sum((base_e + ranks[1]) * oh1, axis=1, keepdims=True)
    p0_ref[...] = pos0.astype(jnp.int32)
    p1_ref[...] = pos1.astype(jnp.int32)
    ends = base_e + padded
    bs = lax.broadcasted_iota(jnp.int32, (GPAD, 1), 0).astype(jnp.float32) * BLK
    be = jnp.sum((bs >= ends).astype(jnp.int32), axis=1, keepdims=True)
    be_ref[...] = jnp.minimum(be, E - 1)


def _route(hidden, ln2, gate_w):
    return pl.pallas_call(
        _route_body,
        grid=(1,),
        in_specs=[
            pl.BlockSpec((S, D), lambda i: (0, 0)),
            pl.BlockSpec((1, D), lambda i: (0, 0)),
            pl.BlockSpec((D, E), lambda i: (0, 0)),
        ],
        out_specs=[
            pl.BlockSpec((S, D), lambda i: (0, 0)),
            pl.BlockSpec((S, E), lambda i: (0, 0)),
            pl.BlockSpec((S, 1), lambda i: (0, 0)),
            pl.BlockSpec((S, 1), lambda i: (0, 0)),
            pl.BlockSpec((S, 1), lambda i: (0, 0)),
            pl.BlockSpec((S, 1), lambda i: (0, 0)),
            pl.BlockSpec((GPAD, 1), lambda i: (0, 0)),
        ],
        out_shape=[
            jax.ShapeDtypeStruct((S, D), jnp.float32),
            jax.ShapeDtypeStruct((S, E), jnp.float32),
            jax.ShapeDtypeStruct((S, 1), jnp.int32),
            jax.ShapeDtypeStruct((S, 1), jnp.int32),
            jax.ShapeDtypeStruct((S, 1), jnp.float32),
            jax.ShapeDtypeStruct((S, 1), jnp.float32),
            jax.ShapeDtypeStruct((GPAD, 1), jnp.int32),
        ],
    )(hidden, ln2, gate_w)


def _dispatch(x2, posf):
    mesh = plsc.VectorSubcoreMesh(core_axis_name="c", subcore_axis_name="s")
    apt = (TOPK * S) // NW

    @functools.partial(
        pl.kernel, mesh=mesh,
        out_type=jax.ShapeDtypeStruct((NSLOT, D), jnp.float32),
        scratch_types=[
            pltpu.VMEM((CHA,), jnp.int32),
            pltpu.VMEM((CHA, D), jnp.float32),
            pltpu.SemaphoreType.DMA,
        ],
    )
    def k(x2_hbm, pos_hbm, xs_hbm, idx_v, rows_v, sem):
        wid = lax.axis_index("s") * _NC + lax.axis_index("c")
        for c in range(apt // CHA):
            a0 = wid * apt + c * CHA
            t0 = lax.rem(a0, S)
            pltpu.sync_copy(x2_hbm.at[pl.ds(t0, CHA)], rows_v)
            pltpu.sync_copy(pos_hbm.at[pl.ds(a0, CHA)], idx_v)
            pltpu.async_copy(rows_v, xs_hbm.at[idx_v], sem).wait()

    return k(x2, posf)


def _mlp_body(be_ref, x_ref, w1_ref, w3_ref, w2_ref, y_ref):
    x = x_ref[...].astype(jnp.bfloat16)
    a = jnp.dot(x, w1_ref[0], preferred_element_type=jnp.float32)
    g = jnp.dot(x, w3_ref[0], preferred_element_type=jnp.float32)
    hcur = (a * jax.nn.sigmoid(a) * g).astype(jnp.bfloat16)
    y_ref[...] = jnp.dot(hcur, w2_ref[0], preferred_element_type=jnp.float32)


def _mlp(be, xs, w1, w3, w2):
    grid_spec = pltpu.PrefetchScalarGridSpec(
        num_scalar_prefetch=1,
        grid=(G,),
        in_specs=[
            pl.BlockSpec((BLK, D), lambda b, be: (b, 0)),
            pl.BlockSpec((1, D, FF), lambda b, be: (be[b], 0, 0)),
            pl.BlockSpec((1, D, FF), lambda b, be: (be[b], 0, 0)),
            pl.BlockSpec((1, FF, D), lambda b, be: (be[b], 0, 0)),
        ],
        out_specs=pl.BlockSpec((BLK, D), lambda b, be: (b, 0)),
    )
    return pl.pallas_call(
        _mlp_body,
        grid_spec=grid_spec,
        out_shape=jax.ShapeDtypeStruct((NSLOT, D), jnp.float32),
    )(be, xs, w1, w3, w2)


def _combine_gather(ys, p0, p1):
    mesh = plsc.VectorSubcoreMesh(core_axis_name="c", subcore_axis_name="s")
    tpt = S // NW

    @functools.partial(
        pl.kernel, mesh=mesh,
        out_type=(jax.ShapeDtypeStruct((S, D), jnp.float32),
                  jax.ShapeDtypeStruct((S, D), jnp.float32)),
        scratch_types=[
            pltpu.VMEM((CHB,), jnp.int32),
            pltpu.VMEM((CHB, D), jnp.float32),
            pltpu.SemaphoreType.DMA,
        ],
    )
    def k(ys_hbm, p0_hbm, p1_hbm, g0_hbm, g1_hbm, idx_v, rows_v, sem):
        wid = lax.axis_index("s") * _NC + lax.axis_index("c")
        for c in range(tpt // CHB):
            t0 = wid * tpt + c * CHB
            pltpu.sync_copy(p0_hbm.at[pl.ds(t0, CHB)], idx_v)
            pltpu.async_copy(ys_hbm.at[idx_v], rows_v, sem).wait()
            pltpu.sync_copy(rows_v, g0_hbm.at[pl.ds(t0, CHB)])
            pltpu.sync_copy(p1_hbm.at[pl.ds(t0, CHB)], idx_v)
            pltpu.async_copy(ys_hbm.at[idx_v], rows_v, sem).wait()
            pltpu.sync_copy(rows_v, g1_hbm.at[pl.ds(t0, CHB)])

    return k(ys, p0, p1)


def _final_body(h_ref, w0_ref, g0_ref, w1_ref, g1_ref, o_ref):
    o_ref[...] = (h_ref[...] + w0_ref[...] * g0_ref[...]
                  + w1_ref[...] * g1_ref[...])


def _final(hidden, w0, g0, w1, g1):
    return pl.pallas_call(
        _final_body,
        grid=(S // SB,),
        in_specs=[
            pl.BlockSpec((SB, D), lambda i: (i, 0)),
            pl.BlockSpec((SB, 1), lambda i: (i, 0)),
            pl.BlockSpec((SB, D), lambda i: (i, 0)),
            pl.BlockSpec((SB, 1), lambda i: (i, 0)),
            pl.BlockSpec((SB, D), lambda i: (i, 0)),
        ],
        out_specs=pl.BlockSpec((SB, D), lambda i: (i, 0)),
        out_shape=jax.ShapeDtypeStruct((S, D), jnp.float32),
    )(hidden, w0, g0, w1, g1)


def kernel(hidden_state, ln1_w, ln2_w, wq, wk, wv, wo, gate_w, w1, w3, w2):
    x = hidden_state.reshape(S, D)
    q, k, v = _qkv(x, ln1_w.reshape(1, D),
                   wq.astype(jnp.bfloat16), wk.astype(jnp.bfloat16),
                   wv.astype(jnp.bfloat16))
    attn = _attention(q, k, v)
    hidden = _oproj(attn, wo.astype(jnp.bfloat16), x)
    x2, logits, p0, p1, w0c, w1c, be = _route(hidden, ln2_w.reshape(1, D),
                                              gate_w)
    posf = jnp.concatenate([p0[:, 0], p1[:, 0]], axis=0)
    xs = _dispatch(x2, posf)
    ys = _mlp(be[:G, 0], xs, _cast_bf16(w1), _cast_bf16(w3), _cast_bf16(w2))
    g0, g1 = _combine_gather(ys, p0[:, 0], p1[:, 0])
    out = _final(hidden, w0c, g0, w1c, g1)
    return out.reshape(B, S, D), logits.reshape(B, S, E)


# speedup vs baseline: 1.2677x; 1.0221x over previous
import functools

import jax
import jax.numpy as jnp
from jax import lax
from jax.experimental import pallas as pl
from jax.experimental.pallas import tpu as pltpu
from jax.experimental.pallas import tpu_sc as plsc

B, S, D = 1, 2048, 1024
H, KV, HD = 16, 8, 64
E, TOPK, FF = 8, 2, 2048
EPS = 1e-6
THETA = 1e6
SB = 256
SCALE = 1.0 / (HD ** 0.5)
BLK = 256
NSLOT = 6144
G = NSLOT // BLK
GPAD = 128
CHA = 32
CHB = 32

_NC, _NS = 2, 16
NW = _NC * _NS


def _qkv_body(x_ref, ln1_ref, wq_ref, wk_ref, wv_ref, q_ref, k_ref, v_ref):
    i = pl.program_id(0)
    x = x_ref[...]
    var = jnp.mean(x * x, axis=1, keepdims=True)
    xb = (x * lax.rsqrt(var + EPS) * ln1_ref[...]).astype(jnp.bfloat16)
    q = jnp.dot(xb, wq_ref[...], preferred_element_type=jnp.float32)
    k = jnp.dot(xb, wk_ref[...], preferred_element_type=jnp.float32)
    v = jnp.dot(xb, wv_ref[...], preferred_element_type=jnp.float32)
    half = HD // 2
    t = (i * SB + lax.broadcasted_iota(jnp.int32, (SB, half), 0)).astype(jnp.float32)
    j2 = lax.broadcasted_iota(jnp.int32, (SB, half), 1).astype(jnp.float32)
    inv = jnp.exp(j2 * (-jnp.log(jnp.float32(THETA)) / half))
    fr = t * inv
    cosf = jnp.concatenate([jnp.cos(fr), jnp.cos(fr)], axis=1)
    sinf = jnp.concatenate([jnp.sin(fr), jnp.sin(fr)], axis=1)
    for h in range(H):
        qh = q[:, h * HD:(h + 1) * HD]
        qr = jnp.concatenate([-qh[:, half:], qh[:, :half]], axis=1)
        q_ref[h] = (qh * cosf + qr * sinf).astype(jnp.bfloat16)
    for h in range(KV):
        kh = k[:, h * HD:(h + 1) * HD]
        kr = jnp.concatenate([-kh[:, half:], kh[:, :half]], axis=1)
        k_ref[h] = (kh * cosf + kr * sinf).astype(jnp.bfloat16)
        v_ref[h] = v[:, h * HD:(h + 1) * HD].astype(jnp.bfloat16)


def _qkv(x, ln1, wq, wk, wv):
    return pl.pallas_call(
        _qkv_body,
        grid=(S // SB,),
        in_specs=[
            pl.BlockSpec((SB, D), lambda i: (i, 0)),
            pl.BlockSpec((1, D), lambda i: (0, 0)),
            pl.BlockSpec((D, H * HD), lambda i: (0, 0)),
            pl.BlockSpec((D, KV * HD), lambda i: (0, 0)),
            pl.BlockSpec((D, KV * HD), lambda i: (0, 0)),
        ],
        out_specs=[
            pl.BlockSpec((H, SB, HD), lambda i: (0, i, 0)),
            pl.BlockSpec((KV, SB, HD), lambda i: (0, i, 0)),
            pl.BlockSpec((KV, SB, HD), lambda i: (0, i, 0)),
        ],
        out_shape=[
            jax.ShapeDtypeStruct((H, S, HD), jnp.bfloat16),
            jax.ShapeDtypeStruct((KV, S, HD), jnp.bfloat16),
            jax.ShapeDtypeStruct((KV, S, HD), jnp.bfloat16),
        ],
        name="qkv_rope",
    )(x, ln1, wq, wk, wv)


def _attn_step(q, kj, vj, m, l, acc, masked):
    s = lax.dot_general(q, kj, (((1,), (1,)), ((), ())),
                        preferred_element_type=jnp.float32) * SCALE
    if masked:
        r = lax.rem(lax.broadcasted_iota(jnp.int32, (2 * SB, SB), 0), SB)
        c = lax.broadcasted_iota(jnp.int32, (2 * SB, SB), 1)
        s = jnp.where(r >= c, s, -jnp.inf)
    mnew = jnp.maximum(m, jnp.max(s, axis=1, keepdims=True))
    p = jnp.exp(s - mnew)
    corr = jnp.exp(m - mnew)
    lnew = l * corr + jnp.sum(p, axis=1, keepdims=True)
    accnew = acc * corr + jnp.dot(p.astype(jnp.bfloat16), vj,
                                  preferred_element_type=jnp.float32)
    return mnew, lnew, accnew


def _attn_body(q_ref, k_ref, v_ref, o_ref):
    i = pl.program_id(1)
    q = jnp.concatenate([q_ref[0], q_ref[1]], axis=0)

    def step(j, carry):
        m, l, acc = carry
        kj = k_ref[0, pl.ds(j * SB, SB), :]
        vj = v_ref[0, pl.ds(j * SB, SB), :]
        return _attn_step(q, kj, vj, m, l, acc, masked=False)

    m0 = jnp.full((2 * SB, 1), -jnp.inf, jnp.float32)
    l0 = jnp.zeros((2 * SB, 1), jnp.float32)
    a0 = jnp.zeros((2 * SB, HD), jnp.float32)
    m, l, acc = lax.fori_loop(0, i, step, (m0, l0, a0))
    kj = k_ref[0, pl.ds(i * SB, SB), :]
    vj = v_ref[0, pl.ds(i * SB, SB), :]
    m, l, acc = _attn_step(q, kj, vj, m, l, acc, masked=True)
    o = (acc / l).astype(jnp.bfloat16)
    o_ref[...] = jnp.concatenate([o[:SB], o[SB:]], axis=1)


def _attention(q, k, v):
    return pl.pallas_call(
        _attn_body,
        grid=(KV, S // SB),
        in_specs=[
            pl.BlockSpec((H // KV, SB, HD), lambda h, i: (h, i, 0)),
            pl.BlockSpec((1, S, HD), lambda h, i: (h, 0, 0)),
            pl.BlockSpec((1, S, HD), lambda h, i: (h, 0, 0)),
        ],
        out_specs=pl.BlockSpec((SB, (H // KV) * HD), lambda h, i: (i, h)),
        out_shape=jax.ShapeDtypeStruct((S, H * HD), jnp.bfloat16),
        name="flash_attn",
    )(q, k, v)


def _oproj_body(a_ref, wo_ref, res_ref, o_ref):
    o_ref[...] = res_ref[...] + jnp.dot(a_ref[...], wo_ref[...],
                                        preferred_element_type=jnp.float32)


def _oproj(attn, wo, res):
    return pl.pallas_call(
        _oproj_body,
        grid=(S // SB,),
        in_specs=[
            pl.BlockSpec((SB, H * HD), lambda i: (i, 0)),
            pl.BlockSpec((H * HD, D), lambda i: (0, 0)),
            pl.BlockSpec((SB, D), lambda i: (i, 0)),
        ],
        out_specs=pl.BlockSpec((SB, D), lambda i: (i, 0)),
        out_shape=jax.ShapeDtypeStruct((S, D), jnp.float32),
        name="oproj",
    )(attn, wo, res)


_CC = 4


def _cast_body(a_ref, b_ref, c_ref, ao_ref, bo_ref, co_ref):
    ao_ref[0] = a_ref[0].astype(jnp.bfloat16)
    bo_ref[0] = b_ref[0].astype(jnp.bfloat16)
    co_ref[0] = c_ref[0].astype(jnp.bfloat16)


def _cast_bf16_all(w1, w3, w2):
    spec13 = pl.BlockSpec((1, D // _CC, FF), lambda i, j: (i, j, 0))
    spec2 = pl.BlockSpec((1, FF // _CC, D), lambda i, j: (i, j, 0))
    return pl.pallas_call(
        _cast_body,
        grid=(E, _CC),
        in_specs=[spec13, spec13, spec2],
        out_specs=[spec13, spec13, spec2],
        out_shape=[
            jax.ShapeDtypeStruct((E, D, FF), jnp.bfloat16),
            jax.ShapeDtypeStruct((E, D, FF), jnp.bfloat16),
            jax.ShapeDtypeStruct((E, FF, D), jnp.bfloat16),
        ],
        name="wcast",
    )(w1, w3, w2)


def _route_body(h_ref, ln2_ref, gw_ref,
                x2_ref, lg_ref, p0_ref, p1_ref, w0_ref, w1_ref, be_ref):
    hx = h_ref[...]
    var = jnp.mean(hx * hx, axis=1, keepdims=True)
    x2 = hx * lax.rsqrt(var + EPS) * ln2_ref[...]
    x2_ref[...] = x2
    logits = jnp.dot(x2, gw_ref[...], preferred_element_type=jnp.float32)
    lg_ref[...] = logits
    mx = jnp.max(logits, axis=1, keepdims=True)
    ex = jnp.exp(logits - mx)
    probs = ex / jnp.sum(ex, axis=1, keepdims=True)
    lane = lax.broadcasted_iota(jnp.int32, (S, E), 1)
    v1 = jnp.max(probs, axis=1, keepdims=True)
    i1 = jnp.min(jnp.where(probs == v1, lane, E), axis=1, keepdims=True)
    probs2 = jnp.where(lane == i1, -1.0, probs)
    v2 = jnp.max(probs2, axis=1, keepdims=True)
    i2 = jnp.min(jnp.where(probs2 == v2, lane, E), axis=1, keepdims=True)
    tot = v1 + v2
    w0_ref[...] = v1 / tot
    w1_ref[...] = v2 / tot
    oh0 = (lane == i1).astype(jnp.float32)
    oh1 = (lane == i2).astype(jnp.float32)
    CH = 256
    tril = (lax.broadcasted_iota(jnp.int32, (CH, CH), 1)
            < lax.broadcasted_iota(jnp.int32, (CH, CH), 0)).astype(jnp.float32)
    base = jnp.zeros((1, E), jnp.float32)
    ranks = []
    for oh in (oh0, oh1):
        parts = []
        for c in range(S // CH):
            blk = oh[c * CH:(c + 1) * CH]
            parts.append(jnp.dot(tril, blk, preferred_element_type=jnp.float32)
                         + base)
            base = base + jnp.sum(blk, axis=0, keepdims=True)
        ranks.append(jnp.concatenate(parts, axis=0))
    cnt = base
    padded = jnp.ceil(cnt / BLK) * BLK
    sut = (lax.broadcasted_iota(jnp.int32, (E, E), 0)
           < lax.broadcasted_iota(jnp.int32, (E, E), 1)).astype(jnp.float32)
    base_e = jnp.dot(padded, sut, preferred_element_type=jnp.float32)
    pos0 = jnp.sum((base_e + ranks[0]) * oh0, axis=1, keepdims=True)
    pos1 = jnp.sum((base_e + ranks[1]) * oh1, axis=1, keepdims=True)
    p0_ref[...] = pos0.astype(jnp.int32)
    p1_ref[...] = pos1.astype(jnp.int32)
    ends = base_e + padded
    bs = lax.broadcasted_iota(jnp.int32, (GPAD, 1), 0).astype(jnp.float32) * BLK
    be = jnp.sum((bs >= ends).astype(jnp.int32), axis=1, keepdims=True)
    be_ref[...] = jnp.minimum(be, E - 1)


def _route(hidden, ln2, gate_w):
    return pl.pallas_call(
        _route_body,
        grid=(1,),
        in_specs=[
            pl.BlockSpec((S, D), lambda i: (0, 0)),
            pl.BlockSpec((1, D), lambda i: (0, 0)),
            pl.BlockSpec((D, E), lambda i: (0, 0)),
        ],
        out_specs=[
            pl.BlockSpec((S, D), lambda i: (0, 0)),
            pl.BlockSpec((S, E), lambda i: (0, 0)),
            pl.BlockSpec((S, 1), lambda i: (0, 0)),
            pl.BlockSpec((S, 1), lambda i: (0, 0)),
            pl.BlockSpec((S, 1), lambda i: (0, 0)),
            pl.BlockSpec((S, 1), lambda i: (0, 0)),
            pl.BlockSpec((GPAD, 1), lambda i: (0, 0)),
        ],
        out_shape=[
            jax.ShapeDtypeStruct((S, D), jnp.float32),
            jax.ShapeDtypeStruct((S, E), jnp.float32),
            jax.ShapeDtypeStruct((S, 1), jnp.int32),
            jax.ShapeDtypeStruct((S, 1), jnp.int32),
            jax.ShapeDtypeStruct((S, 1), jnp.float32),
            jax.ShapeDtypeStruct((S, 1), jnp.float32),
            jax.ShapeDtypeStruct((GPAD, 1), jnp.int32),
        ],
        name="route",
    )(hidden, ln2, gate_w)


def _dispatch(x2, posf):
    mesh = plsc.VectorSubcoreMesh(core_axis_name="c", subcore_axis_name="s")
    apt = (TOPK * S) // NW

    @functools.partial(
        pl.kernel, mesh=mesh, name="sc_dispatch",
        out_type=jax.ShapeDtypeStruct((NSLOT, D), jnp.float32),
        scratch_types=[
            pltpu.VMEM((CHA,), jnp.int32),
            pltpu.VMEM((CHA, D), jnp.float32),
            pltpu.SemaphoreType.DMA,
        ],
    )
    def k(x2_hbm, pos_hbm, xs_hbm, idx_v, rows_v, sem):
        wid = lax.axis_index("s") * _NC + lax.axis_index("c")
        for c in range(apt // CHA):
            a0 = wid * apt + c * CHA
            t0 = lax.rem(a0, S)
            pltpu.sync_copy(x2_hbm.at[pl.ds(t0, CHA)], rows_v)
            pltpu.sync_copy(pos_hbm.at[pl.ds(a0, CHA)], idx_v)
            pltpu.async_copy(rows_v, xs_hbm.at[idx_v], sem).wait()

    return k(x2, posf)


def _mlp_body(be_ref, x_ref, w1_ref, w3_ref, w2_ref, y_ref):
    x = x_ref[...].astype(jnp.bfloat16)
    a = jnp.dot(x, w1_ref[0], preferred_element_type=jnp.float32)
    g = jnp.dot(x, w3_ref[0], preferred_element_type=jnp.float32)
    hcur = (a * jax.nn.sigmoid(a) * g).astype(jnp.bfloat16)
    y_ref[...] = jnp.dot(hcur, w2_ref[0], preferred_element_type=jnp.float32)


def _mlp(be, xs, w1, w3, w2):
    grid_spec = pltpu.PrefetchScalarGridSpec(
        num_scalar_prefetch=1,
        grid=(G,),
        in_specs=[
            pl.BlockSpec((BLK, D), lambda b, be: (b, 0)),
            pl.BlockSpec((1, D, FF), lambda b, be: (be[b], 0, 0)),
            pl.BlockSpec((1, D, FF), lambda b, be: (be[b], 0, 0)),
            pl.BlockSpec((1, FF, D), lambda b, be: (be[b], 0, 0)),
        ],
        out_specs=pl.BlockSpec((BLK, D), lambda b, be: (b, 0)),
    )
    return pl.pallas_call(
        _mlp_body,
        grid_spec=grid_spec,
        out_shape=jax.ShapeDtypeStruct((NSLOT, D), jnp.float32),
        name="moe_mlp",
    )(be, xs, w1, w3, w2)


def _combine_gather(ys, p0, p1):
    mesh = plsc.VectorSubcoreMesh(core_axis_name="c", subcore_axis_name="s")
    tpt = S // NW

    @functools.partial(
        pl.kernel, mesh=mesh, name="sc_combine",
        out_type=(jax.ShapeDtypeStruct((S, D), jnp.float32),
                  jax.ShapeDtypeStruct((S, D), jnp.float32)),
        scratch_types=[
            pltpu.VMEM((CHB,), jnp.int32),
            pltpu.VMEM((CHB, D), jnp.float32),
            pltpu.SemaphoreType.DMA,
        ],
    )
    def k(ys_hbm, p0_hbm, p1_hbm, g0_hbm, g1_hbm, idx_v, rows_v, sem):
        wid = lax.axis_index("s") * _NC + lax.axis_index("c")
        for c in range(tpt // CHB):
            t0 = wid * tpt + c * CHB
            pltpu.sync_copy(p0_hbm.at[pl.ds(t0, CHB)], idx_v)
            pltpu.async_copy(ys_hbm.at[idx_v], rows_v, sem).wait()
            pltpu.sync_copy(rows_v, g0_hbm.at[pl.ds(t0, CHB)])
            pltpu.sync_copy(p1_hbm.at[pl.ds(t0, CHB)], idx_v)
            pltpu.async_copy(ys_hbm.at[idx_v], rows_v, sem).wait()
            pltpu.sync_copy(rows_v, g1_hbm.at[pl.ds(t0, CHB)])

    return k(ys, p0, p1)


def _final_body(h_ref, w0_ref, g0_ref, w1_ref, g1_ref, o_ref):
    o_ref[...] = (h_ref[...] + w0_ref[...] * g0_ref[...]
                  + w1_ref[...] * g1_ref[...])


def _final(hidden, w0, g0, w1, g1):
    return pl.pallas_call(
        _final_body,
        grid=(S // SB,),
        in_specs=[
            pl.BlockSpec((SB, D), lambda i: (i, 0)),
            pl.BlockSpec((SB, 1), lambda i: (i, 0)),
            pl.BlockSpec((SB, D), lambda i: (i, 0)),
            pl.BlockSpec((SB, 1), lambda i: (i, 0)),
            pl.BlockSpec((SB, D), lambda i: (i, 0)),
        ],
        out_specs=pl.BlockSpec((SB, D), lambda i: (i, 0)),
        out_shape=jax.ShapeDtypeStruct((S, D), jnp.float32),
        name="combine_add",
    )(hidden, w0, g0, w1, g1)


def kernel(hidden_state, ln1_w, ln2_w, wq, wk, wv, wo, gate_w, w1, w3, w2):
    x = hidden_state.reshape(S, D)
    w1b, w3b, w2b = _cast_bf16_all(w1, w3, w2)
    q, k, v = _qkv(x, ln1_w.reshape(1, D),
                   wq.astype(jnp.bfloat16), wk.astype(jnp.bfloat16),
                   wv.astype(jnp.bfloat16))
    attn = _attention(q, k, v)
    hidden = _oproj(attn, wo.astype(jnp.bfloat16), x)
    x2, logits, p0, p1, w0c, w1c, be = _route(hidden, ln2_w.reshape(1, D),
                                              gate_w)
    posf = jnp.concatenate([p0[:, 0], p1[:, 0]], axis=0)
    xs = _dispatch(x2, posf)
    ys = _mlp(be[:G, 0], xs, w1b, w3b, w2b)
    g0, g1 = _combine_gather(ys, p0[:, 0], p1[:, 0])
    out = _final(hidden, w0c, g0, w1c, g1)
    return out.reshape(B, S, D), logits.reshape(B, S, E)
